# initial kernel scaffold (unmeasured)
import jax
import jax.numpy as jnp
from jax import lax
from jax.experimental import pallas as pl
from jax.experimental.pallas import tpu as pltpu

N = 32
SQ = 512
DM = 1024
HQ = 8
DH = 128
SKV = 2048
CH = SQ // N
CROWS = CH + 1
SCALE = 0.08838834764831843

_sem_signal = getattr(pl, "semaphore_signal", None) or pltpu.semaphore_signal
_sem_wait = getattr(pl, "semaphore_wait", None) or pltpu.semaphore_wait
_MESH = (getattr(pl, "DeviceIdType", None) or pltpu.DeviceIdType).MESH


def _body(x_ref, wq_ref, wo_ref, k_ref, v_ref, out_ref,
          acc_ref, comm_ref, agbuf_ref,
          rs_send, rs_recv, ag_send, ag_recv, rs_credit, ag_credit):
    my = lax.axis_index("i")
    right = lax.rem(my + 1, N)
    left = lax.rem(my + N - 1, N)

    q = jnp.dot(x_ref[:, :], wq_ref[:, :], preferred_element_type=jnp.float32)
    outs = []
    lsums = []
    for h in range(HQ):
        qh = q[:, h * DH:(h + 1) * DH]
        kh = k_ref[h, :, :]
        s_mat = lax.dot_general(
            qh, kh, (((1,), (1,)), ((), ())),
            preferred_element_type=jnp.float32) * SCALE
        p = jnp.exp(s_mat)
        lsums.append(jnp.sum(p, axis=1))
        outs.append(jnp.dot(p, v_ref[h, :, :],
                            preferred_element_type=jnp.float32))
    o_all = jnp.concatenate(outs, axis=1)
    l_all = jnp.stack(lsums, axis=1)

    o3 = o_all.reshape(N, CH, DM)
    lp = l_all.reshape(N, 1, CH * HQ)
    lp = jnp.concatenate(
        [lp, jnp.zeros((N, 1, DM - CH * HQ), jnp.float32)], axis=2)
    acc_ref[:, :, :] = jnp.concatenate([o3, lp], axis=1)

    for s in range(N - 1):
        if s >= 2:
            _sem_wait(rs_credit, 1)
        cs = lax.rem(my - s + N, N)
        rdma = pltpu.make_async_remote_copy(
            src_ref=acc_ref.at[cs],
            dst_ref=comm_ref.at[s % 2],
            send_sem=rs_send.at[s % 2],
            recv_sem=rs_recv.at[s % 2],
            device_id=(right,),
            device_id_type=_MESH,
        )
        rdma.start()
        rdma.wait()
        cr = lax.rem(my - s - 1 + N, N)
        cur = acc_ref[pl.ds(cr, 1), :, :]
        acc_ref[pl.ds(cr, 1), :, :] = cur + comm_ref[pl.ds(s % 2, 1), :, :]
        if s <= N - 4:
            _sem_signal(rs_credit, inc=1, device_id=(left,),
                        device_id_type=_MESH)

    r = lax.rem(my + 1, N)
    block = acc_ref[pl.ds(r, 1), :, :].reshape(CROWS, DM)
    o = block[0:CH, :].reshape(CH, HQ, DH)
    l_chunk = block[CH, 0:CH * HQ].reshape(CH, HQ)
    y = (o / l_chunk[:, :, None]).reshape(CH, DM)
    y = jnp.dot(y, wo_ref[:, :], preferred_element_type=jnp.float32)
    out_ref[pl.ds(r * CH, CH), :] = y

    for s in range(N - 1):
        if s >= 2:
            _sem_wait(ag_credit, 1)
        a = lax.rem(my + 1 - s + N, N)
        rdma = pltpu.make_async_remote_copy(
            src_ref=out_ref.at[pl.ds(a * CH, CH), :],
            dst_ref=agbuf_ref.at[s % 2],
            send_sem=ag_send.at[s % 2],
            recv_sem=ag_recv.at[s % 2],
            device_id=(right,),
            device_id_type=_MESH,
        )
        rdma.start()
        rdma.wait()
        g = lax.rem(my - s + N, N)
        out_ref[pl.ds(g * CH, CH), :] = agbuf_ref[s % 2, :, :]
        if s <= N - 4:
            _sem_signal(ag_credit, inc=1, device_id=(left,),
                        device_id_type=_MESH)


def kernel(x, Wq, Wo, K_ext, V_ext):
    xs = x[0]
    k = jnp.transpose(K_ext[0], (1, 0, 2))
    v = jnp.transpose(V_ext[0], (1, 0, 2))

    out = pl.pallas_call(
        _body,
        out_shape=jax.ShapeDtypeStruct((SQ, DM), jnp.float32),
        in_specs=[pl.BlockSpec(memory_space=pltpu.VMEM)] * 5,
        out_specs=pl.BlockSpec(memory_space=pltpu.VMEM),
        scratch_shapes=[
            pltpu.VMEM((N, CROWS, DM), jnp.float32),
            pltpu.VMEM((2, CROWS, DM), jnp.float32),
            pltpu.VMEM((2, CH, DM), jnp.float32),
            pltpu.SemaphoreType.DMA((2,)),
            pltpu.SemaphoreType.DMA((2,)),
            pltpu.SemaphoreType.DMA((2,)),
            pltpu.SemaphoreType.DMA((2,)),
            pltpu.SemaphoreType.REGULAR,
            pltpu.SemaphoreType.REGULAR,
        ],
    )(xs, Wq, Wo, k, v)
    return out.reshape(1, SQ, DM)


# baseline (device time: 284080 ns/iter reference)
import jax
import jax.numpy as jnp
from jax import lax
from jax.experimental import pallas as pl
from jax.experimental.pallas import tpu as pltpu

N = 32
SQ = 512
DM = 1024
HQ = 8
DH = 128
SKV = 2048
CH = SQ // N
DME = DM + 128
SCALE = 0.08838834764831843

_sem_signal = getattr(pl, "semaphore_signal", None) or pltpu.semaphore_signal
_sem_wait = getattr(pl, "semaphore_wait", None) or pltpu.semaphore_wait
_MESH = (getattr(pl, "DeviceIdType", None) or pltpu.DeviceIdType).MESH


def _body(x_ref, wq_ref, wo_ref, k_ref, v_ref, out_ref,
          acc_ref, comm_ref, agbuf_ref,
          rs_send, rs_recv, ag_send, ag_recv, rs_credit, ag_credit):
    my = lax.axis_index("i")
    right = lax.rem(my + 1, N)
    left = lax.rem(my + N - 1, N)

    q = jnp.dot(x_ref[:, :], wq_ref[:, :], preferred_element_type=jnp.float32)
    outs = []
    lsums = []
    for h in range(HQ):
        qh = q[:, h * DH:(h + 1) * DH]
        kh = k_ref[h, :, :]
        s_mat = lax.dot_general(
            qh, kh, (((1,), (1,)), ((), ())),
            preferred_element_type=jnp.float32) * SCALE
        p = jnp.exp(s_mat)
        lsums.append(jnp.sum(p, axis=1))
        outs.append(jnp.dot(p, v_ref[h, :, :],
                            preferred_element_type=jnp.float32))
    o_all = jnp.concatenate(outs, axis=1)
    l_all = jnp.stack(lsums, axis=1)
    l_pad = jnp.concatenate(
        [l_all, jnp.zeros((SQ, DME - DM - HQ), jnp.float32)], axis=1)
    full = jnp.concatenate([o_all, l_pad], axis=1)
    acc_ref[:, :, :] = full.reshape(N, CH, DME)

    for s in range(N - 1):
        if s >= 2:
            _sem_wait(rs_credit, 1)
        cs = lax.rem(my - s + N, N)
        rdma = pltpu.make_async_remote_copy(
            src_ref=acc_ref.at[cs],
            dst_ref=comm_ref.at[s % 2],
            send_sem=rs_send.at[s % 2],
            recv_sem=rs_recv.at[s % 2],
            device_id=(right,),
            device_id_type=_MESH,
        )
        rdma.start()
        rdma.wait()
        cr = lax.rem(my - s - 1 + N, N)
        cur = acc_ref[pl.ds(cr, 1), :, :]
        acc_ref[pl.ds(cr, 1), :, :] = cur + comm_ref[pl.ds(s % 2, 1), :, :]
        if s <= N - 4:
            _sem_signal(rs_credit, inc=1, device_id=(left,),
                        device_id_type=_MESH)

    r = lax.rem(my + 1, N)
    block = acc_ref[pl.ds(r, 1), :, :].reshape(CH, DME)
    o = block[:, 0:DM].reshape(CH, HQ, DH)
    l_chunk = block[:, DM:DM + HQ]
    y = (o / l_chunk[:, :, None]).reshape(CH, DM)
    y = jnp.dot(y, wo_ref[:, :], preferred_element_type=jnp.float32)
    out_ref[pl.ds(r * CH, CH), :] = y

    for s in range(N - 1):
        if s >= 2:
            _sem_wait(ag_credit, 1)
        a = lax.rem(my + 1 - s + N, N)
        rdma = pltpu.make_async_remote_copy(
            src_ref=out_ref.at[pl.ds(a * CH, CH), :],
            dst_ref=agbuf_ref.at[s % 2],
            send_sem=ag_send.at[s % 2],
            recv_sem=ag_recv.at[s % 2],
            device_id=(right,),
            device_id_type=_MESH,
        )
        rdma.start()
        rdma.wait()
        g = lax.rem(my - s + N, N)
        out_ref[pl.ds(g * CH, CH), :] = agbuf_ref[s % 2, :, :]
        if s <= N - 4:
            _sem_signal(ag_credit, inc=1, device_id=(left,),
                        device_id_type=_MESH)


def kernel(x, Wq, Wo, K_ext, V_ext):
    xs = x[0]
    k = jnp.transpose(K_ext[0], (1, 0, 2))
    v = jnp.transpose(V_ext[0], (1, 0, 2))

    out = pl.pallas_call(
        _body,
        out_shape=jax.ShapeDtypeStruct((SQ, DM), jnp.float32),
        in_specs=[pl.BlockSpec(memory_space=pltpu.VMEM)] * 5,
        out_specs=pl.BlockSpec(memory_space=pltpu.VMEM),
        scratch_shapes=[
            pltpu.VMEM((N, CH, DME), jnp.float32),
            pltpu.VMEM((2, CH, DME), jnp.float32),
            pltpu.VMEM((2, CH, DM), jnp.float32),
            pltpu.SemaphoreType.DMA((2,)),
            pltpu.SemaphoreType.DMA((2,)),
            pltpu.SemaphoreType.DMA((2,)),
            pltpu.SemaphoreType.DMA((2,)),
            pltpu.SemaphoreType.REGULAR,
            pltpu.SemaphoreType.REGULAR,
        ],
    )(xs, Wq, Wo, k, v)
    return out.reshape(1, SQ, DM)


# device time: 195527 ns/iter; 1.4529x vs baseline; 1.4529x over previous
import jax
import jax.numpy as jnp
from jax import lax
from jax.experimental import pallas as pl
from jax.experimental.pallas import tpu as pltpu

N = 32
SQ = 512
DM = 1024
HQ = 8
DH = 128
CH = SQ // N
DME = DM + 128
SCALE = 0.08838834764831843

CW_HOPS = N // 2
CCW_HOPS = N // 2 - 1

_sem_signal = getattr(pl, "semaphore_signal", None) or pltpu.semaphore_signal
_sem_wait = getattr(pl, "semaphore_wait", None) or pltpu.semaphore_wait
_MESH = (getattr(pl, "DeviceIdType", None) or pltpu.DeviceIdType).MESH


def _body(x_ref, wq_ref, wo_ref, k_ref, v_ref, out_ref,
          acc_ref, cwbuf_ref, ccwbuf_ref, agcw_ref, agccw_ref,
          cw_send, cw_recv, ccw_send, ccw_recv,
          agcw_send, agcw_recv, agccw_send, agccw_recv,
          cw_credit, ccw_credit, agcw_credit, agccw_credit):
    my = lax.axis_index("i")
    right = lax.rem(my + 1, N)
    left = lax.rem(my + N - 1, N)

    q = jnp.dot(x_ref[:, :], wq_ref[:, :], preferred_element_type=jnp.float32)
    outs = []
    lsums = []
    for h in range(HQ):
        qh = q[:, h * DH:(h + 1) * DH]
        kh = k_ref[h, :, :]
        s_mat = lax.dot_general(
            qh, kh, (((1,), (1,)), ((), ())),
            preferred_element_type=jnp.float32) * SCALE
        p = jnp.exp(s_mat)
        lsums.append(jnp.sum(p, axis=1))
        outs.append(jnp.dot(p, v_ref[h, :, :],
                            preferred_element_type=jnp.float32))
    o_all = jnp.concatenate(outs, axis=1)
    l_all = jnp.stack(lsums, axis=1)
    l_pad = jnp.concatenate(
        [l_all, jnp.zeros((SQ, DME - DM - HQ), jnp.float32)], axis=1)
    full = jnp.concatenate([o_all, l_pad], axis=1)
    acc_ref[:, :, :] = full.reshape(N, CH, DME)

    for s in range(CW_HOPS):
        if s >= 2:
            _sem_wait(cw_credit, 1)
            if s <= CCW_HOPS - 1:
                _sem_wait(ccw_credit, 1)
        c_cw = lax.rem(my + 17 - s + N, N)
        rdma_cw = pltpu.make_async_remote_copy(
            src_ref=acc_ref.at[c_cw],
            dst_ref=cwbuf_ref.at[s % 2],
            send_sem=cw_send.at[s % 2],
            recv_sem=cw_recv.at[s % 2],
            device_id=(right,),
            device_id_type=_MESH,
        )
        rdma_cw.start()
        rdma_ccw = None
        if s <= CCW_HOPS - 1:
            c_ccw = lax.rem(my - 14 + s + N, N)
            rdma_ccw = pltpu.make_async_remote_copy(
                src_ref=acc_ref.at[c_ccw],
                dst_ref=ccwbuf_ref.at[s % 2],
                send_sem=ccw_send.at[s % 2],
                recv_sem=ccw_recv.at[s % 2],
                device_id=(left,),
                device_id_type=_MESH,
            )
            rdma_ccw.start()
        rdma_cw.wait()
        r_cw = lax.rem(my + 16 - s + N, N)
        acc_ref[pl.ds(r_cw, 1), :, :] = (
            acc_ref[pl.ds(r_cw, 1), :, :]
            + cwbuf_ref[pl.ds(s % 2, 1), :, :])
        if s <= CW_HOPS - 3:
            _sem_signal(cw_credit, inc=1, device_id=(left,),
                        device_id_type=_MESH)
        if rdma_ccw is not None:
            rdma_ccw.wait()
            r_ccw = lax.rem(my - 13 + s + N, N)
            acc_ref[pl.ds(r_ccw, 1), :, :] = (
                acc_ref[pl.ds(r_ccw, 1), :, :]
                + ccwbuf_ref[pl.ds(s % 2, 1), :, :])
            if s <= CCW_HOPS - 3:
                _sem_signal(ccw_credit, inc=1, device_id=(right,),
                            device_id_type=_MESH)

    r = lax.rem(my + 1, N)
    block = acc_ref[pl.ds(r, 1), :, :].reshape(CH, DME)
    o = block[:, 0:DM].reshape(CH, HQ, DH)
    l_chunk = block[:, DM:DM + HQ]
    y = (o / l_chunk[:, :, None]).reshape(CH, DM)
    y = jnp.dot(y, wo_ref[:, :], preferred_element_type=jnp.float32)
    out_ref[pl.ds(r * CH, CH), :] = y

    for s in range(CW_HOPS):
        if s >= 2:
            _sem_wait(agcw_credit, 1)
            if s <= CCW_HOPS - 1:
                _sem_wait(agccw_credit, 1)
        a_cw = lax.rem(my + 1 - s + N, N)
        rdma_cw = pltpu.make_async_remote_copy(
            src_ref=out_ref.at[pl.ds(a_cw * CH, CH), :],
            dst_ref=agcw_ref.at[s % 2],
            send_sem=agcw_send.at[s % 2],
            recv_sem=agcw_recv.at[s % 2],
            device_id=(right,),
            device_id_type=_MESH,
        )
        rdma_cw.start()
        rdma_ccw = None
        if s <= CCW_HOPS - 1:
            a_ccw = lax.rem(my + 1 + s, N)
            rdma_ccw = pltpu.make_async_remote_copy(
                src_ref=out_ref.at[pl.ds(a_ccw * CH, CH), :],
                dst_ref=agccw_ref.at[s % 2],
                send_sem=agccw_send.at[s % 2],
                recv_sem=agccw_recv.at[s % 2],
                device_id=(left,),
                device_id_type=_MESH,
            )
            rdma_ccw.start()
        rdma_cw.wait()
        g_cw = lax.rem(my - s + N, N)
        out_ref[pl.ds(g_cw * CH, CH), :] = agcw_ref[s % 2, :, :]
        if s <= CW_HOPS - 3:
            _sem_signal(agcw_credit, inc=1, device_id=(left,),
                        device_id_type=_MESH)
        if rdma_ccw is not None:
            rdma_ccw.wait()
            g_ccw = lax.rem(my + 2 + s, N)
            out_ref[pl.ds(g_ccw * CH, CH), :] = agccw_ref[s % 2, :, :]
            if s <= CCW_HOPS - 3:
                _sem_signal(agccw_credit, inc=1, device_id=(right,),
                            device_id_type=_MESH)


def kernel(x, Wq, Wo, K_ext, V_ext):
    xs = x[0]
    k = jnp.transpose(K_ext[0], (1, 0, 2))
    v = jnp.transpose(V_ext[0], (1, 0, 2))

    out = pl.pallas_call(
        _body,
        out_shape=jax.ShapeDtypeStruct((SQ, DM), jnp.float32),
        in_specs=[pl.BlockSpec(memory_space=pltpu.VMEM)] * 5,
        out_specs=pl.BlockSpec(memory_space=pltpu.VMEM),
        scratch_shapes=[
            pltpu.VMEM((N, CH, DME), jnp.float32),
            pltpu.VMEM((2, CH, DME), jnp.float32),
            pltpu.VMEM((2, CH, DME), jnp.float32),
            pltpu.VMEM((2, CH, DM), jnp.float32),
            pltpu.VMEM((2, CH, DM), jnp.float32),
            pltpu.SemaphoreType.DMA((2,)),
            pltpu.SemaphoreType.DMA((2,)),
            pltpu.SemaphoreType.DMA((2,)),
            pltpu.SemaphoreType.DMA((2,)),
            pltpu.SemaphoreType.DMA((2,)),
            pltpu.SemaphoreType.DMA((2,)),
            pltpu.SemaphoreType.DMA((2,)),
            pltpu.SemaphoreType.DMA((2,)),
            pltpu.SemaphoreType.REGULAR,
            pltpu.SemaphoreType.REGULAR,
            pltpu.SemaphoreType.REGULAR,
            pltpu.SemaphoreType.REGULAR,
        ],
    )(xs, Wq, Wo, k, v)
    return out.reshape(1, SQ, DM)
